# baseline (device time: 249789 ns/iter reference)
import jax
import jax.numpy as jnp
from jax import lax
from jax.experimental import pallas as pl
from jax.experimental.pallas import tpu as pltpu

B, SQ, H, D = 4, 32, 8, 128
BK = 256


def kernel(Q, K, V):
    skv = K.shape[1]
    nsteps = skv // BK
    scale = D ** -0.5

    def body(q_ref, k_ref, v_ref, o_ref,
             acc, m, l, acc_rx, stats_tx, stats_rx,
             sem_o_s, sem_o_r, sem_s_s, sem_s_r):
        i = pl.program_id(0)

        @pl.when(i == 0)
        def _init():
            m[...] = jnp.full(m.shape, -jnp.inf, jnp.float32)
            l[...] = jnp.zeros(l.shape, jnp.float32)
            acc[...] = jnp.zeros(acc.shape, jnp.float32)

        for bb in range(B):
            for hh in range(H):
                q = q_ref[bb, :, hh, :].astype(jnp.bfloat16)
                k = k_ref[bb, :, hh, :].astype(jnp.bfloat16)
                v = v_ref[bb, :, hh, :].astype(jnp.bfloat16)
                s = lax.dot_general(
                    q, k, (((1,), (1,)), ((), ())),
                    preferred_element_type=jnp.float32) * scale
                m_cur = jnp.max(s, axis=-1, keepdims=True)
                m_old = m[bb, hh]
                m_new = jnp.maximum(m_old, m_cur)
                alpha = jnp.exp(m_old - m_new)
                p = jnp.exp(s - m_new)
                l[bb, hh] = l[bb, hh] * alpha + jnp.sum(p, axis=-1, keepdims=True)
                pv = lax.dot_general(
                    p.astype(jnp.bfloat16), v, (((1,), (0,)), ((), ())),
                    preferred_element_type=jnp.float32)
                acc[bb, hh] = acc[bb, hh] * alpha + pv
                m[bb, hh] = m_new

        @pl.when(i == nsteps - 1)
        def _exchange_and_merge():
            my_x = lax.axis_index("x")
            my_y = lax.axis_index("y")
            my_z = lax.axis_index("z")
            partner = (my_x, 1 - my_y, my_z)

            stats_tx[0] = m[...]
            stats_tx[1] = l[...]
            rdma_o = pltpu.make_async_remote_copy(
                src_ref=acc, dst_ref=acc_rx,
                send_sem=sem_o_s, recv_sem=sem_o_r,
                device_id=partner, device_id_type=pl.DeviceIdType.MESH)
            rdma_s = pltpu.make_async_remote_copy(
                src_ref=stats_tx, dst_ref=stats_rx,
                send_sem=sem_s_s, recv_sem=sem_s_r,
                device_id=partner, device_id_type=pl.DeviceIdType.MESH)
            rdma_o.start()
            rdma_s.start()
            rdma_o.wait()
            rdma_s.wait()

            for bb in range(B):
                for hh in range(H):
                    m_l = m[bb, hh]
                    m_r = stats_rx[0, bb, hh]
                    m_n = jnp.maximum(m_l, m_r)
                    a_l = jnp.exp(m_l - m_n)
                    a_r = jnp.exp(m_r - m_n)
                    l_n = l[bb, hh] * a_l + stats_rx[1, bb, hh] * a_r
                    o = (acc[bb, hh] * a_l + acc_rx[bb, hh] * a_r) / l_n
                    o_ref[bb, :, hh, :] = o

    return pl.pallas_call(
        body,
        grid=(nsteps,),
        in_specs=[
            pl.BlockSpec((B, SQ, H, D), lambda i: (0, 0, 0, 0)),
            pl.BlockSpec((B, BK, H, D), lambda i: (0, i, 0, 0)),
            pl.BlockSpec((B, BK, H, D), lambda i: (0, i, 0, 0)),
        ],
        out_specs=pl.BlockSpec((B, SQ, H, D), lambda i: (0, 0, 0, 0)),
        out_shape=jax.ShapeDtypeStruct((B, SQ, H, D), jnp.float32),
        scratch_shapes=[
            pltpu.VMEM((B, H, SQ, D), jnp.float32),
            pltpu.VMEM((B, H, SQ, 1), jnp.float32),
            pltpu.VMEM((B, H, SQ, 1), jnp.float32),
            pltpu.VMEM((B, H, SQ, D), jnp.float32),
            pltpu.VMEM((2, B, H, SQ, 1), jnp.float32),
            pltpu.VMEM((2, B, H, SQ, 1), jnp.float32),
            pltpu.SemaphoreType.DMA,
            pltpu.SemaphoreType.DMA,
            pltpu.SemaphoreType.DMA,
            pltpu.SemaphoreType.DMA,
        ],
        compiler_params=pltpu.CompilerParams(
            dimension_semantics=("arbitrary",)),
    )(Q, K, V)


# device time: 202974 ns/iter; 1.2306x vs baseline; 1.2306x over previous
import jax
import jax.numpy as jnp
from jax import lax
from jax.experimental import pallas as pl
from jax.experimental.pallas import tpu as pltpu

B, SQ, H, D = 4, 32, 8, 128


def kernel(Q, K, V):
    skv = K.shape[1]
    scale = D ** -0.5

    Q = Q.reshape(B, SQ, H * D)
    K = K.reshape(B, skv, H * D)
    V = V.reshape(B, skv, H * D)

    def body(q_ref, k_ref, v_ref, o_ref,
             acc, acc_rx, stats_tx, stats_rx,
             o_send, o_recv, s_send, s_recv):
        h = pl.program_id(0)

        def partner_rdma(hh):
            my_x = lax.axis_index("x")
            my_y = lax.axis_index("y")
            my_z = lax.axis_index("z")
            partner = (my_x, 1 - my_y, my_z)
            rdma_o = pltpu.make_async_remote_copy(
                src_ref=acc.at[hh], dst_ref=acc_rx.at[hh],
                send_sem=o_send.at[hh], recv_sem=o_recv.at[hh],
                device_id=partner, device_id_type=pl.DeviceIdType.MESH)
            rdma_s = pltpu.make_async_remote_copy(
                src_ref=stats_tx.at[hh], dst_ref=stats_rx.at[hh],
                send_sem=s_send.at[hh], recv_sem=s_recv.at[hh],
                device_id=partner, device_id_type=pl.DeviceIdType.MESH)
            return rdma_o, rdma_s

        for bb in range(B):
            q = q_ref[bb].astype(jnp.bfloat16)
            k = k_ref[bb].astype(jnp.bfloat16)
            v = v_ref[bb].astype(jnp.bfloat16)
            s = lax.dot_general(
                q, k, (((1,), (1,)), ((), ())),
                preferred_element_type=jnp.float32) * scale
            m_c = jnp.max(s, axis=-1, keepdims=True)
            p = jnp.exp(s - m_c)
            l_c = jnp.sum(p, axis=-1, keepdims=True)
            pv = lax.dot_general(
                p.astype(jnp.bfloat16), v, (((1,), (0,)), ((), ())),
                preferred_element_type=jnp.float32)
            acc[h, bb] = pv
            stats_tx[h, 0, bb] = m_c
            stats_tx[h, 1, bb] = l_c

        rdma_o, rdma_s = partner_rdma(h)
        rdma_o.start()
        rdma_s.start()

        @pl.when(h == H - 1)
        def _merge():
            for hh in range(H):
                w_o, w_s = partner_rdma(hh)
                w_o.wait()
                w_s.wait()
            m_l = stats_tx[:, 0]
            l_l = stats_tx[:, 1]
            m_r = stats_rx[:, 0]
            l_r = stats_rx[:, 1]
            m_n = jnp.maximum(m_l, m_r)
            a_l = jnp.exp(m_l - m_n)
            a_r = jnp.exp(m_r - m_n)
            l_n = l_l * a_l + l_r * a_r
            o = (acc[...] * a_l + acc_rx[...] * a_r) / l_n
            for hh in range(H):
                o_ref[:, :, hh * D:(hh + 1) * D] = o[hh]

    out = pl.pallas_call(
        body,
        grid=(H,),
        in_specs=[
            pl.BlockSpec((B, SQ, D), lambda h: (0, 0, h)),
            pl.BlockSpec((B, skv, D), lambda h: (0, 0, h)),
            pl.BlockSpec((B, skv, D), lambda h: (0, 0, h)),
        ],
        out_specs=pl.BlockSpec((B, SQ, H * D), lambda h: (0, 0, 0)),
        out_shape=jax.ShapeDtypeStruct((B, SQ, H * D), jnp.float32),
        scratch_shapes=[
            pltpu.VMEM((H, B, SQ, D), jnp.float32),
            pltpu.VMEM((H, B, SQ, D), jnp.float32),
            pltpu.VMEM((H, 2, B, SQ, 1), jnp.float32),
            pltpu.VMEM((H, 2, B, SQ, 1), jnp.float32),
            pltpu.SemaphoreType.DMA((H,)),
            pltpu.SemaphoreType.DMA((H,)),
            pltpu.SemaphoreType.DMA((H,)),
            pltpu.SemaphoreType.DMA((H,)),
        ],
        compiler_params=pltpu.CompilerParams(
            dimension_semantics=("arbitrary",),
            vmem_limit_bytes=64 * 1024 * 1024),
    )(Q, K, V)
    return out.reshape(B, SQ, H, D)
